# baseline (speedup 1.0000x reference)
import functools

import jax
import jax.numpy as jnp
from jax.experimental import pallas as pl
from jax.experimental.pallas import tpu as pltpu
from jax.experimental.pallas import tpu_sc as plsc

_K = 1024
_D = 64
_BLK = 2048


def _vq_body(x_ref, cb_ref, xsq_ref, csq_ref, idx_ref):
    xb = x_ref[...]
    cb = cb_ref[...]
    mm = jax.lax.dot_general(xb, cb, (((1,), (1,)), ((), ())),
                             preferred_element_type=jnp.float32)
    x_sq = xsq_ref[...]
    c_sq = csq_ref[...]
    d2 = x_sq + c_sq - 2.0 * mm
    dist = jnp.sqrt(jnp.maximum(d2, 0.0))
    m = jnp.min(dist, axis=1, keepdims=True)
    lane = jax.lax.broadcasted_iota(jnp.int32, (_BLK, _K), 1)
    idx = jnp.min(jnp.where(dist == m, lane, _K), axis=1).astype(jnp.int32)
    idx_ref[...] = idx


@functools.partial(jax.jit, static_argnames=("interpret",))
def _vq_indices(flat_x, codebook, interpret=False):
    n = flat_x.shape[0]
    xsq = jnp.sum(flat_x * flat_x, axis=1, keepdims=True)
    csq = jnp.sum(codebook * codebook, axis=1)[None, :]
    return pl.pallas_call(
        _vq_body,
        grid=(n // _BLK,),
        in_specs=[
            pl.BlockSpec((_BLK, _D), lambda i: (i, 0)),
            pl.BlockSpec((_K, _D), lambda i: (0, 0)),
            pl.BlockSpec((_BLK, 1), lambda i: (i, 0)),
            pl.BlockSpec((1, _K), lambda i: (0, 0)),
        ],
        out_specs=pl.BlockSpec((_BLK,), lambda i: (i,)),
        out_shape=jax.ShapeDtypeStruct((n,), jnp.int32),
        interpret=interpret,
    )(flat_x, codebook, xsq, csq)


def _sc_gather(codebook, idx_flat):
    n = idx_flat.shape[0]
    mesh = plsc.VectorSubcoreMesh(core_axis_name="c", subcore_axis_name="s")
    nc = mesh.num_cores
    nw = nc * mesh.num_subcores
    b_per_w = n // nw

    @functools.partial(
        pl.kernel,
        out_type=jax.ShapeDtypeStruct((n, _D), jnp.float32),
        mesh=mesh,
        scratch_types=[
            pltpu.VMEM((b_per_w,), jnp.int32),
            pltpu.VMEM((b_per_w, _D), jnp.float32),
            pltpu.SemaphoreType.DMA,
        ],
        compiler_params=pltpu.CompilerParams(use_tc_tiling_on_sc=False),
    )
    def gk(cb_hbm, idx_hbm, out_hbm, idx_v, rows_v, sem):
        wid = jax.lax.axis_index("s") * nc + jax.lax.axis_index("c")
        base = wid * b_per_w
        pltpu.sync_copy(idx_hbm.at[pl.ds(base, b_per_w)], idx_v)
        pltpu.async_copy(cb_hbm.at[idx_v], rows_v, sem).wait()
        pltpu.sync_copy(rows_v, out_hbm.at[pl.ds(base, b_per_w)])

    return gk(codebook, idx_flat)


def kernel(x, codebook):
    input_shape = x.shape
    flat_x = x.reshape(-1, codebook.shape[1])
    idx = _vq_indices(flat_x, codebook)
    q = _sc_gather(codebook, idx)
    return idx.reshape(input_shape[:-1]), q.reshape(input_shape)


# speedup vs baseline: 1.0317x; 1.0317x over previous
import functools

import jax
import jax.numpy as jnp
from jax.experimental import pallas as pl
from jax.experimental.pallas import tpu as pltpu
from jax.experimental.pallas import tpu_sc as plsc

_K = 1024
_D = 64
_BLK = 2048


def _vq_body(x_ref, cb_ref, xsq_ref, csq_ref, idx_ref):
    xb = x_ref[...]
    cb = cb_ref[...]
    mm = jax.lax.dot_general(xb, cb, (((1,), (1,)), ((), ())),
                             preferred_element_type=jnp.float32)
    x_sq = xsq_ref[...]
    c_sq = csq_ref[...]
    d2 = x_sq + c_sq - 2.0 * mm
    dist = jnp.sqrt(jnp.maximum(d2, 0.0))
    m = jnp.min(dist, axis=1, keepdims=True)
    lane = jax.lax.broadcasted_iota(jnp.int32, (_BLK, _K), 1)
    idx = jnp.min(jnp.where(dist == m, lane, _K), axis=1).astype(jnp.int32)
    idx_ref[...] = idx


@functools.partial(jax.jit, static_argnames=("interpret",))
def _vq_indices(flat_x, codebook, interpret=False):
    n = flat_x.shape[0]
    xsq = jnp.sum(flat_x * flat_x, axis=1, keepdims=True)
    csq = jnp.sum(codebook * codebook, axis=1)[None, :]
    return pl.pallas_call(
        _vq_body,
        grid=(n // _BLK,),
        in_specs=[
            pl.BlockSpec((_BLK, _D), lambda i: (i, 0)),
            pl.BlockSpec((_K, _D), lambda i: (0, 0)),
            pl.BlockSpec((_BLK, 1), lambda i: (i, 0)),
            pl.BlockSpec((1, _K), lambda i: (0, 0)),
        ],
        out_specs=pl.BlockSpec((_BLK,), lambda i: (i,)),
        out_shape=jax.ShapeDtypeStruct((n,), jnp.int32),
        interpret=interpret,
    )(flat_x, codebook, xsq, csq)


def _sc_gather(codebook_pad, idx_flat):
    n = idx_flat.shape[0]
    mesh = plsc.VectorSubcoreMesh(core_axis_name="c", subcore_axis_name="s")
    nc = mesh.num_cores
    nw = nc * mesh.num_subcores
    b_per_w = n // nw

    @functools.partial(
        pl.kernel,
        out_type=jax.ShapeDtypeStruct((n, 128), jnp.float32),
        mesh=mesh,
        scratch_types=[
            pltpu.VMEM((b_per_w // 2,), jnp.int32),
            pltpu.VMEM((b_per_w // 2, 128), jnp.float32),
            pltpu.SemaphoreType.DMA,
        ],
    )
    def gk(cb_hbm, idx_hbm, out_hbm, idx_v, rows_v, sem):
        wid = jax.lax.axis_index("s") * nc + jax.lax.axis_index("c")
        half = b_per_w // 2
        for h in range(2):
            base = wid * b_per_w + h * half
            pltpu.sync_copy(idx_hbm.at[pl.ds(base, half)], idx_v)
            pltpu.async_copy(cb_hbm.at[idx_v], rows_v, sem).wait()
            pltpu.sync_copy(rows_v, out_hbm.at[pl.ds(base, half)])

    return gk(codebook_pad, idx_flat)


def kernel(x, codebook):
    input_shape = x.shape
    flat_x = x.reshape(-1, codebook.shape[1])
    idx = _vq_indices(flat_x, codebook)
    cb_pad = jnp.pad(codebook, ((0, 0), (0, 128 - _D)))
    q = _sc_gather(cb_pad, idx)[:, :_D]
    return idx.reshape(input_shape[:-1]), q.reshape(input_shape)


# speedup vs baseline: 1.0932x; 1.0597x over previous
import functools

import jax
import jax.numpy as jnp
from jax.experimental import pallas as pl
from jax.experimental.pallas import tpu as pltpu
from jax.experimental.pallas import tpu_sc as plsc

_K = 1024
_D = 64
_BLK = 2048


def _vq_body(x_ref, cb_ref, xsq_ref, csq_ref, idx_ref):
    xb = x_ref[...]
    cb = cb_ref[...]
    mm = jax.lax.dot_general(xb, cb, (((1,), (1,)), ((), ())),
                             preferred_element_type=jnp.float32)
    x_sq = xsq_ref[...]
    c_sq = csq_ref[...]
    d2 = x_sq + c_sq - 2.0 * mm
    d2c = jnp.maximum(d2, 0.0)
    dist = jnp.where(d2c == 0.0, 0.0, d2c * jax.lax.rsqrt(d2c))
    m = jnp.min(dist, axis=1, keepdims=True)
    lane = jax.lax.broadcasted_iota(jnp.int32, (_BLK, _K), 1)
    idx = jnp.min(jnp.where(dist == m, lane, _K), axis=1).astype(jnp.int32)
    idx_ref[...] = idx


@functools.partial(jax.jit, static_argnames=("interpret",))
def _vq_indices(flat_x, codebook, interpret=False):
    n = flat_x.shape[0]
    xsq = jnp.sum(flat_x * flat_x, axis=1, keepdims=True)
    csq = jnp.sum(codebook * codebook, axis=1)[None, :]
    return pl.pallas_call(
        _vq_body,
        grid=(n // _BLK,),
        in_specs=[
            pl.BlockSpec((_BLK, _D), lambda i: (i, 0)),
            pl.BlockSpec((_K, _D), lambda i: (0, 0)),
            pl.BlockSpec((_BLK, 1), lambda i: (i, 0)),
            pl.BlockSpec((1, _K), lambda i: (0, 0)),
        ],
        out_specs=pl.BlockSpec((_BLK,), lambda i: (i,)),
        out_shape=jax.ShapeDtypeStruct((n,), jnp.int32),
        interpret=interpret,
    )(flat_x, codebook, xsq, csq)


def _sc_gather(codebook_pad, idx_flat):
    n = idx_flat.shape[0]
    mesh = plsc.VectorSubcoreMesh(core_axis_name="c", subcore_axis_name="s")
    nc = mesh.num_cores
    nw = nc * mesh.num_subcores
    b_per_w = n // nw

    @functools.partial(
        pl.kernel,
        out_type=jax.ShapeDtypeStruct((n, 128), jnp.float32),
        mesh=mesh,
        scratch_types=[
            pltpu.VMEM((b_per_w // 2,), jnp.int32),
            pltpu.VMEM((b_per_w // 2, 128), jnp.float32),
            pltpu.SemaphoreType.DMA,
        ],
    )
    def gk(cb_hbm, idx_hbm, out_hbm, idx_v, rows_v, sem):
        wid = jax.lax.axis_index("s") * nc + jax.lax.axis_index("c")
        half = b_per_w // 2
        for h in range(2):
            base = wid * b_per_w + h * half
            pltpu.sync_copy(idx_hbm.at[pl.ds(base, half)], idx_v)
            pltpu.async_copy(cb_hbm.at[idx_v], rows_v, sem).wait()
            pltpu.sync_copy(rows_v, out_hbm.at[pl.ds(base, half)])

    return gk(codebook_pad, idx_flat)


def kernel(x, codebook):
    input_shape = x.shape
    flat_x = x.reshape(-1, codebook.shape[1])
    idx = _vq_indices(flat_x, codebook)
    cb_pad = jnp.pad(codebook, ((0, 0), (0, 128 - _D)))
    q = _sc_gather(cb_pad, idx)[:, :_D]
    return idx.reshape(input_shape[:-1]), q.reshape(input_shape)
